# baseline (speedup 1.0000x reference)
import functools
import math

import jax
import jax.numpy as jnp
from jax import lax
from jax.experimental import pallas as pl
from jax.experimental.pallas import tpu as pltpu
from jax.experimental.pallas import tpu_sc as plsc

D_MODEL = 64
SCALE = math.sqrt(D_MODEL)
NUM_CORES = 2
NUM_SUBCORES = 16
NUM_WORKERS = NUM_CORES * NUM_SUBCORES
LANES = 16
BLK = 128
PITCH = BLK + 17
NBUF = 4


def _idx_stage(n_batch: int, n_seq: int):
  n_sh = n_seq // 8
  n_bh = n_batch // BLK
  n_tiles = n_sh * n_bh
  per_w = n_tiles // NUM_WORKERS
  assert n_tiles % NUM_WORKERS == 0
  mesh = plsc.VectorSubcoreMesh(core_axis_name="c", subcore_axis_name="s")

  @functools.partial(
      pl.kernel,
      mesh=mesh,
      out_type=jax.ShapeDtypeStruct((n_tiles, 8, BLK), jnp.int32),
      scratch_types=[
          pltpu.VMEM((per_w, 8, BLK), jnp.int32),
          pltpu.SemaphoreType.DMA,
          pltpu.SemaphoreType.DMA,
      ],
  )
  def ks(idxT_hbm, out_hbm, buf, rsem, wsem):
    wid = lax.axis_index("s") * NUM_CORES + lax.axis_index("c")
    t0 = wid * per_w
    for i in range(per_w):
      t = t0 + i
      sh = t // n_bh
      bh = t % n_bh
      pltpu.async_copy(
          idxT_hbm.at[pl.ds(sh * 8, 8), pl.ds(bh * BLK, BLK)],
          buf.at[i], rsem)
    for i in range(per_w):
      pltpu.make_async_copy(idxT_hbm.at[pl.ds(0, 8), pl.ds(0, BLK)],
                            buf.at[0], rsem).wait()
    for i in range(per_w):
      pltpu.async_copy(buf.at[i], out_hbm.at[t0 + i], wsem)
    for i in range(per_w):
      pltpu.make_async_copy(buf.at[0], out_hbm.at[0], wsem).wait()

  return ks


def _embed_kernel(n_batch: int, n_seq: int):
  n_total = n_batch * n_seq
  n_bblk = n_batch // BLK
  n_tasks = n_seq * n_bblk
  tasks_per_w = n_tasks // NUM_WORKERS
  idx_per_w = n_total // NUM_WORKERS
  assert n_tasks % NUM_WORKERS == 0 and tasks_per_w % NBUF == 0
  mesh = plsc.VectorSubcoreMesh(core_axis_name="c", subcore_axis_name="s")

  @functools.partial(
      pl.kernel,
      mesh=mesh,
      out_type=jax.ShapeDtypeStruct(
          (n_seq, D_MODEL // 8, n_bblk, 8, BLK), jnp.float32),
      scratch_types=[
          pltpu.VMEM((idx_per_w,), jnp.int32),
          *[pltpu.VMEM((BLK, D_MODEL), jnp.float32) for _ in range(NBUF)],
          *[pltpu.VMEM((D_MODEL, PITCH), jnp.float32) for _ in range(NBUF)],
          *[pltpu.SemaphoreType.DMA for _ in range(2 * NBUF)],
      ],
      compiler_params=pltpu.CompilerParams(
          use_tc_tiling_on_sc=False, needs_layout_passes=False),
  )
  def k(idx_hbm, table_hbm, out_hbm, idx_v,
        r0, r1, r2, r3, t0, t1, t2, t3,
        g0, g1, g2, g3, w0, w1, w2, w3):
    rows = [r0, r1, r2, r3]
    tiles = [t0, t1, t2, t3]
    gsem = [g0, g1, g2, g3]
    wsem = [w0, w1, w2, w3]
    wid = lax.axis_index("s") * NUM_CORES + lax.axis_index("c")
    base_task = wid * tasks_per_w
    pltpu.sync_copy(idx_hbm.at[pl.ds(wid * idx_per_w, idx_per_w)], idx_v)

    def start_gather(k_local, b):
      off = pl.multiple_of(k_local * BLK, 8)
      pltpu.async_copy(table_hbm.at[idx_v.at[pl.ds(off, BLK)]],
                       rows[b], gsem[b])

    def wait_gather(b):
      pltpu.make_async_copy(table_hbm.at[idx_v.at[pl.ds(0, BLK)]],
                            rows[b], gsem[b]).wait()

    def start_writes(k_local, b):
      tau = base_task + k_local
      sh = tau // (n_bblk * 8)
      bh = (tau // 8) % n_bblk
      sl = tau % 8
      s = sh * 8 + sl
      for ch in range(D_MODEL // 8):
        pltpu.async_copy(tiles[b].at[pl.ds(ch * 8, 8), pl.ds(0, BLK)],
                         out_hbm.at[s, ch, bh], wsem[b])

    def wait_writes(b):
      for ch in range(D_MODEL // 8):
        pltpu.make_async_copy(tiles[b].at[pl.ds(ch * 8, 8), pl.ds(0, BLK)],
                              out_hbm.at[0, ch, 0], wsem[b]).wait()

    iota = lax.iota(jnp.int32, LANES)
    cvecs = [iota + (j * LANES) for j in range(D_MODEL // LANES)]

    def transpose_scale(b):
      def tbody(r, carry):
        rvec = jnp.full((LANES,), 0, jnp.int32) + r
        for j in range(D_MODEL // LANES):
          v = rows[b][r, pl.ds(j * LANES, LANES)]
          plsc.store_scatter(tiles[b], [cvecs[j], rvec], v * SCALE)
        return carry

      lax.fori_loop(0, BLK, tbody, 0, unroll=8)
      return None

    start_gather(0, 0)
    start_gather(1, 1)

    def group(g, carry):
      for b in range(NBUF):
        kk = g * NBUF + b
        wait_gather(b)

        @pl.when(kk >= 1)
        def _():
          start_writes(kk - 1, (b - 1) % NBUF)

        @pl.when(kk >= NBUF)
        def _():
          wait_writes(b)

        transpose_scale(b)

        @pl.when(kk < tasks_per_w - 2)
        def _():
          start_gather(kk + 2, (b + 2) % NBUF)
      return carry

    lax.fori_loop(0, tasks_per_w // NBUF, group, 0)
    plsc.subcore_barrier()
    start_writes(tasks_per_w - 1, (tasks_per_w - 1) % NBUF)
    for b in range(NBUF):
      wait_writes(b)

  return k


def kernel(input, table):
  b0, b1 = input.shape
  idx3 = _idx_stage(b0, b1)(input.T.astype(jnp.int32))
  out5 = _embed_kernel(b0, b1)(idx3.reshape(b0 * b1), table)
  return out5.transpose(2, 4, 0, 1, 3).reshape(b0, b1, D_MODEL)
